# baseline (speedup 1.0000x reference)
import functools

import jax
import jax.numpy as jnp
from jax import lax
from jax.experimental import pallas as pl
from jax.experimental.pallas import tpu as pltpu
from jax.experimental.pallas import tpu_sc as plsc

N = 10000
E = 160000
H = 256
L = 10
NE = 119
NET = 7

NTILES = 32
EP = 163840
EC = EP // NTILES
B = 128
NCHUNK = EC // B
ECS = EP // 16
NCHUNKS = ECS // B
BG = 64
NCB = 40
NPHASE = ECS // (BG * NCB)
NCG = NPHASE * NCB
NP = 10112
NPT = NP // 16

_MESH = plsc.VectorSubcoreMesh(core_axis_name="c", subcore_axis_name="s")


def _sc_prep(ntab, nidx, posx, posy, posz, srcr, dstr):
    @functools.partial(
        pl.kernel,
        mesh=_MESH,
        out_type=[
            jax.ShapeDtypeStruct((20480, 128), jnp.float32),
            jax.ShapeDtypeStruct((NTILES, NCHUNK, B), jnp.float32),
        ],
        scratch_types=[
            pltpu.VMEM((5, 128), jnp.int32),
            pltpu.VMEM((128, 128), jnp.float32),
            pltpu.VMEM((NCHUNK, B), jnp.int32),
            pltpu.VMEM((NCHUNK, B), jnp.int32),
            pltpu.VMEM((6, B), jnp.float32),
            pltpu.VMEM((B,), jnp.float32),
            pltpu.SemaphoreType.DMA,
        ],
    )
    def k(ntab_hbm, nidx_hbm, px_hbm, py_hbm, pz_hbm, srcr_hbm, dstr_hbm,
          h0_out, d2_out, nidx_v, nrows_v, sidx_v, didx_v, xyz_v, d2_v, sem):
        c = lax.axis_index("c")
        s = lax.axis_index("s")
        w = s * 2 + c

        pltpu.sync_copy(nidx_hbm.at[w], nidx_v)
        for j in range(5):
            pltpu.async_copy(ntab_hbm.at[nidx_v.at[j]], nrows_v, sem).wait()
            pltpu.sync_copy(nrows_v, h0_out.at[pl.ds(w * 640 + j * 128, 128)])

        pltpu.sync_copy(srcr_hbm.at[w], sidx_v)
        pltpu.sync_copy(dstr_hbm.at[w], didx_v)

        def chunk(kk, _):
            for comp, p_hbm in enumerate((px_hbm, py_hbm, pz_hbm)):
                pltpu.async_copy(p_hbm.at[sidx_v.at[kk]],
                                 xyz_v.at[comp], sem).wait()
                pltpu.async_copy(p_hbm.at[didx_v.at[kk]],
                                 xyz_v.at[3 + comp], sem).wait()

            def vec(v, _):
                sl = pl.ds(v * 16, 16)
                dx = xyz_v[0, sl] - xyz_v[3, sl]
                dy = xyz_v[1, sl] - xyz_v[4, sl]
                dz = xyz_v[2, sl] - xyz_v[5, sl]
                d2_v[sl] = dx * dx + dy * dy + dz * dz
                return 0

            lax.fori_loop(0, B // 16, vec, 0, unroll=True)
            pltpu.sync_copy(d2_v, d2_out.at[w, kk])
            return 0

        lax.fori_loop(0, NCHUNK, chunk, 0)

    return k(ntab, nidx, posx, posy, posz, srcr, dstr)


def _sc_gine(hcat, ee, gidx, dsta, zeros_np):
    @functools.partial(
        pl.kernel,
        mesh=_MESH,
        out_type=jax.ShapeDtypeStruct((2, NP, 128), jnp.float32),
        scratch_types=[
            pltpu.VMEM((NCB, BG), jnp.int32),
            pltpu.VMEM((NCB, BG), jnp.int32),
            pltpu.VMEM((2, BG, 128), jnp.float32),
            pltpu.VMEM((2, BG, 128), jnp.float32),
            pltpu.VMEM_SHARED((NP, 128), jnp.float32),
            pltpu.SemaphoreType.DMA,
            pltpu.SemaphoreType.DMA,
            pltpu.SemaphoreType.DMA,
            pltpu.SemaphoreType.DMA,
            pltpu.SemaphoreType.DMA,
            pltpu.SemaphoreType.DMA,
        ],
    )
    def k(hcat_hbm, ee_hbm, gidx_hbm, dsta_hbm, zeros_hbm, agg_out,
          gidx_v, dst_v, hbuf, eebuf, aggsh, g0, g1, e0, e1, s0, s1):
        c = lax.axis_index("c")
        s = lax.axis_index("s")

        pltpu.sync_copy(zeros_hbm.at[pl.ds(s * NPT, NPT)],
                        aggsh.at[pl.ds(s * NPT, NPT)])
        plsc.subcore_barrier()

        def issue_g(p, kk, slot, gsem):
            pltpu.async_copy(hcat_hbm.at[gidx_v.at[kk]], hbuf.at[slot], gsem)

        def issue_e(p, kk, slot, esem):
            pltpu.async_copy(
                ee_hbm.at[c, pl.ds(s * ECS + (p * NCB + kk) * BG, BG)],
                eebuf.at[slot], esem)

        def wait_loads(slot, gsem, esem):
            pltpu.make_async_copy(hcat_hbm.at[gidx_v.at[0]],
                                  hbuf.at[slot], gsem).wait()
            pltpu.make_async_copy(ee_hbm.at[c, pl.ds(0, BG)],
                                  eebuf.at[slot], esem).wait()

        def wait_scatter(slot, ssem):
            pltpu.make_async_copy(ee_hbm.at[c, pl.ds(0, BG)],
                                  eebuf.at[slot], ssem).wait()

        def compute_scatter(kk, slot, ssem):
            def edge(e, _):
                for v in range(8):
                    sl = pl.ds(v * 16, 16)
                    eebuf[slot, e, sl] = jnp.maximum(
                        hbuf[slot, e, sl] + eebuf[slot, e, sl], 0.0)
                return 0

            lax.fori_loop(0, BG, edge, 0, unroll=4)
            pltpu.async_copy(eebuf.at[slot], aggsh.at[dst_v.at[kk]],
                             ssem, add=True)

        def phase(p, _):
            @pl.when(p > 0)
            def _():
                wait_scatter(0, s0)
                wait_scatter(1, s1)

            pltpu.sync_copy(gidx_hbm.at[c, s, pl.ds(p * NCB, NCB)], gidx_v)
            pltpu.sync_copy(dsta_hbm.at[s, pl.ds(p * NCB, NCB)], dst_v)
            issue_g(p, 0, 0, g0)
            issue_e(p, 0, 0, e0)
            issue_g(p, 1, 1, g1)

            def pair(j, _):
                @pl.when(j > 0)
                def _():
                    wait_scatter(1, s1)
                issue_e(p, 2 * j + 1, 1, e1)
                wait_loads(0, g0, e0)
                compute_scatter(2 * j, 0, s0)

                @pl.when(j < NCB // 2 - 1)
                def _():
                    issue_g(p, 2 * j + 2, 0, g0)

                @pl.when(j < NCB // 2 - 1)
                def _():
                    wait_scatter(0, s0)
                    issue_e(p, 2 * j + 2, 0, e0)
                wait_loads(1, g1, e1)
                compute_scatter(2 * j + 1, 1, s1)

                @pl.when(j < NCB // 2 - 1)
                def _():
                    issue_g(p, 2 * j + 3, 1, g1)
                return 0

            lax.fori_loop(0, NCB // 2, pair, 0)
            return 0

        lax.fori_loop(0, NPHASE, phase, 0)
        wait_scatter(0, s0)
        wait_scatter(1, s1)
        plsc.subcore_barrier()
        pltpu.sync_copy(aggsh.at[pl.ds(s * NPT, NPT)],
                        agg_out.at[c, pl.ds(s * NPT, NPT)])

    return k(hcat, ee, gidx, dsta, zeros_np)


def _sc_hcomb(hcat, gsrc, gdst):
    @functools.partial(
        pl.kernel,
        mesh=_MESH,
        out_type=jax.ShapeDtypeStruct((2, EP, 128), jnp.float32),
        scratch_types=[
            pltpu.VMEM((NCHUNKS, B), jnp.int32),
            pltpu.VMEM((NCHUNKS, B), jnp.int32),
            pltpu.VMEM((2, B, 128), jnp.float32),
            pltpu.VMEM((2, B, 128), jnp.float32),
            pltpu.VMEM((2, B, 128), jnp.float32),
            pltpu.SemaphoreType.DMA,
            pltpu.SemaphoreType.DMA,
            pltpu.SemaphoreType.DMA,
            pltpu.SemaphoreType.DMA,
            pltpu.SemaphoreType.DMA,
            pltpu.SemaphoreType.DMA,
        ],
    )
    def k(hcat_hbm, gsrc_hbm, gdst_hbm, hc_out,
          gs_v, gd_v, abuf, bbuf, obuf, a0, a1, b0, b1, w0, w1):
        c = lax.axis_index("c")
        s = lax.axis_index("s")
        asems = (a0, a1)
        bsems = (b0, b1)
        wsems = (w0, w1)

        pltpu.sync_copy(gsrc_hbm.at[c, s], gs_v)
        pltpu.sync_copy(gdst_hbm.at[c, s], gd_v)

        def issue(kk, slot):
            pltpu.async_copy(hcat_hbm.at[gs_v.at[kk]], abuf.at[slot],
                             asems[slot])
            pltpu.async_copy(hcat_hbm.at[gd_v.at[kk]], bbuf.at[slot],
                             bsems[slot])

        def wait_loads(slot):
            pltpu.make_async_copy(hcat_hbm.at[gs_v.at[0]], abuf.at[slot],
                                  asems[slot]).wait()
            pltpu.make_async_copy(hcat_hbm.at[gd_v.at[0]], bbuf.at[slot],
                                  bsems[slot]).wait()

        def wait_write(slot):
            pltpu.make_async_copy(hc_out.at[c, pl.ds(0, B)], obuf.at[slot],
                                  wsems[slot]).wait()

        def compute_write(kk, slot):
            def edge(e, _):
                for v in range(8):
                    sl = pl.ds(v * 16, 16)
                    obuf[slot, e, sl] = abuf[slot, e, sl] * bbuf[slot, e, sl]
                return 0

            lax.fori_loop(0, B, edge, 0, unroll=4)
            pltpu.async_copy(obuf.at[slot],
                             hc_out.at[c, pl.ds(s * ECS + kk * B, B)],
                             wsems[slot])

        issue(0, 0)
        issue(1, 1)

        def pair(j, _):
            @pl.when(j > 0)
            def _():
                issue(2 * j + 1, 1)

            wait_loads(0)

            @pl.when(j > 0)
            def _():
                wait_write(0)

            compute_write(2 * j, 0)

            @pl.when(j < NCHUNKS // 2 - 1)
            def _():
                issue(2 * j + 2, 0)

            wait_loads(1)

            @pl.when(j > 0)
            def _():
                wait_write(1)

            compute_write(2 * j + 1, 1)
            return 0

        lax.fori_loop(0, NCHUNKS // 2, pair, 0)
        wait_write(0)
        wait_write(1)

    return k(hcat, gsrc, gdst)


BE2 = 2048


def _tc_edge_prep(dist2, et, etab, dW1, dB1, dW2, dB2):
    def body(d2_ref, et_ref, etab_ref, w1_ref, b1_ref, w2_ref, b2_ref,
             de_ref, ee_ref):
        d = jnp.sqrt(d2_ref[...])
        h1 = jnp.maximum(d * w1_ref[...] + b1_ref[...], 0.0)
        de = jax.lax.dot(h1, w2_ref[...],
                         preferred_element_type=jnp.float32) + b2_ref[...]
        et = et_ref[...]
        sel = jnp.zeros_like(de)
        for t in range(NET):
            sel = sel + jnp.where(et == t, 1.0, 0.0) * etab_ref[t:t + 1, :]
        ee = sel * de
        de_ref[...] = de
        ee_ref[0] = ee[:, :128]
        ee_ref[1] = ee[:, 128:]

    return pl.pallas_call(
        body,
        grid=(EP // BE2,),
        in_specs=[
            pl.BlockSpec((BE2, 1), lambda i: (i, 0)),
            pl.BlockSpec((BE2, 1), lambda i: (i, 0)),
            pl.BlockSpec((8, H), lambda i: (0, 0)),
            pl.BlockSpec((1, H), lambda i: (0, 0)),
            pl.BlockSpec((1, H), lambda i: (0, 0)),
            pl.BlockSpec((H, H), lambda i: (0, 0)),
            pl.BlockSpec((1, H), lambda i: (0, 0)),
        ],
        out_specs=[
            pl.BlockSpec((BE2, H), lambda i: (i, 0)),
            pl.BlockSpec((2, BE2, 128), lambda i: (0, i, 0)),
        ],
        out_shape=[
            jax.ShapeDtypeStruct((EP, H), jnp.float32),
            jax.ShapeDtypeStruct((2, EP, 128), jnp.float32),
        ],
    )(dist2, et, etab, dW1, dB1, dW2, dB2)


BN = 2000


def _tc_gin_mlp(h, agg, w1, b1, w2, b2, eps1):
    def body(h_ref, agg_ref, w1_ref, b1_ref, w2_ref, b2_ref, eps_ref, out_ref):
        e = eps_ref[0, 0]
        x0 = e * h_ref[0] + agg_ref[0]
        x1 = e * h_ref[1] + agg_ref[1]
        hm = jnp.maximum(
            jax.lax.dot(x0, w1_ref[...][:128],
                        preferred_element_type=jnp.float32)
            + jax.lax.dot(x1, w1_ref[...][128:],
                          preferred_element_type=jnp.float32)
            + b1_ref[...], 0.0)
        y = jax.lax.dot(hm, w2_ref[...],
                        preferred_element_type=jnp.float32) + b2_ref[...]
        out_ref[0] = y[:, :128]
        out_ref[1] = y[:, 128:]

    return pl.pallas_call(
        body,
        grid=(N // BN,),
        in_specs=[
            pl.BlockSpec((2, BN, 128), lambda i: (0, i, 0)),
            pl.BlockSpec((2, BN, 128), lambda i: (0, i, 0)),
            pl.BlockSpec((H, H), lambda i: (0, 0)),
            pl.BlockSpec((1, H), lambda i: (0, 0)),
            pl.BlockSpec((H, H), lambda i: (0, 0)),
            pl.BlockSpec((1, H), lambda i: (0, 0)),
            pl.BlockSpec((1, 1), lambda i: (0, 0), memory_space=pltpu.SMEM),
        ],
        out_specs=pl.BlockSpec((2, BN, 128), lambda i: (0, i, 0)),
        out_shape=jax.ShapeDtypeStruct((2, N, 128), jnp.float32),
    )(h, agg, w1, b1, w2, b2, eps1)


def _tc_sigma_prep(sigma1, sW1, sB1, sW2, sB2, oW1c, oB1):
    def body(sg_ref, w1_ref, b1_ref, w2_ref, b2_ref, wc_ref, ob_ref, out_ref):
        s = jnp.log(jnp.maximum(sg_ref[...], 1e-12))
        h1 = jnp.maximum(s * w1_ref[...] + b1_ref[...], 0.0)
        se = jax.lax.dot(h1, w2_ref[...],
                         preferred_element_type=jnp.float32) + b2_ref[...]
        out_ref[...] = ob_ref[...] + jax.lax.dot(
            se, wc_ref[...], preferred_element_type=jnp.float32)

    return pl.pallas_call(
        body,
        out_shape=jax.ShapeDtypeStruct((1, H), jnp.float32),
    )(sigma1, sW1, sB1, sW2, sB2, oW1c, oB1)


BE = 1600


def _tc_final_mlp(hc, de, emf, w1a, w1b, b1eff, oW2, oB2, oW3p, oB3p, sigma1):
    def body(hc_ref, de_ref, emf_ref, w1a_ref, w1b_ref, b1_ref,
             w2_ref, b2_ref, w3_ref, b3_ref, sg_ref, out_ref):
        w1a = w1a_ref[...]
        h1 = jnp.maximum(
            jax.lax.dot(hc_ref[0], w1a[:128], preferred_element_type=jnp.float32)
            + jax.lax.dot(hc_ref[1], w1a[128:], preferred_element_type=jnp.float32)
            + jax.lax.dot(de_ref[...], w1b_ref[...],
                          preferred_element_type=jnp.float32)
            + b1_ref[...], 0.0)
        h2 = jnp.maximum(
            jax.lax.dot(h1, w2_ref[...], preferred_element_type=jnp.float32)
            + b2_ref[...], 0.0)
        raw = jax.lax.dot(h2, w3_ref[...],
                          preferred_element_type=jnp.float32) + b3_ref[...]
        out_ref[...] = raw * emf_ref[...] / sg_ref[0, 0]

    return pl.pallas_call(
        body,
        grid=(E // BE,),
        in_specs=[
            pl.BlockSpec((2, BE, 128), lambda i: (0, i, 0)),
            pl.BlockSpec((BE, H), lambda i: (i, 0)),
            pl.BlockSpec((BE, 1), lambda i: (i, 0)),
            pl.BlockSpec((H, H), lambda i: (0, 0)),
            pl.BlockSpec((H, H), lambda i: (0, 0)),
            pl.BlockSpec((1, H), lambda i: (0, 0)),
            pl.BlockSpec((H, H // 2), lambda i: (0, 0)),
            pl.BlockSpec((1, H // 2), lambda i: (0, 0)),
            pl.BlockSpec((H // 2, 128), lambda i: (0, 0)),
            pl.BlockSpec((1, 128), lambda i: (0, 0)),
            pl.BlockSpec((1, 1), lambda i: (0, 0), memory_space=pltpu.SMEM),
        ],
        out_specs=pl.BlockSpec((BE, 128), lambda i: (i, 0)),
        out_shape=jax.ShapeDtypeStruct((E, 128), jnp.float32),
    )(hc, de, emf, w1a, w1b, b1eff, oW2, oB2, oW3p, oB3p, sigma1)


def kernel(pos, edge_index, atom_type, edge_type, edge_mask, sigma,
           node_table, edge_table, dW1, dB1, dW2, dB2, sW1, sB1, sW2, sB2,
           gin_eps, gW1, gB1, gW2, gB2, oW1, oB1, oW2, oB2, oW3, oB3):
    f32 = jnp.float32
    src, dst = edge_index[0], edge_index[1]
    pad = EP - E
    srcp = jnp.concatenate([src, jnp.zeros((pad,), jnp.int32)])
    dstp = jnp.concatenate([dst, jnp.zeros((pad,), jnp.int32)])
    etp = jnp.concatenate([edge_type, jnp.zeros((pad,), jnp.int32)])
    dst_adj = jnp.concatenate(
        [jnp.where(edge_mask, dst, N), jnp.full((pad,), N, jnp.int32)])

    gidx_src = jnp.stack([srcp, srcp + N]).reshape(2, 16, NCG, BG)
    gidx_dst = jnp.stack([dstp, dstp + N]).reshape(2, 16, NCHUNKS, B)
    gidx_src128 = gidx_src.reshape(2, 16, NCHUNKS, B)
    dsta = dst_adj.reshape(16, NCG, BG)
    srcr = srcp.reshape(NTILES, NCHUNK, B)
    dstr = dstp.reshape(NTILES, NCHUNK, B)
    nidx = jnp.concatenate(
        [atom_type, atom_type + NE,
         jnp.zeros((20480 - 2 * N,), jnp.int32)]).reshape(NTILES, 5, 128)
    ntab_cat = jnp.concatenate([node_table[:, :128], node_table[:, 128:]], 0)
    zeros_np = jnp.zeros((NP, 128), f32)

    h0p, dist2 = _sc_prep(ntab_cat, nidx, pos[:, 0], pos[:, 1], pos[:, 2],
                          srcr, dstr)

    etab = jnp.concatenate([edge_table, jnp.zeros((1, H), f32)], 0)
    de, ee = _tc_edge_prep(dist2.reshape(EP, 1), etp.reshape(EP, 1), etab,
                           dW1, dB1.reshape(1, H), dW2, dB2.reshape(1, H))

    hflat = h0p
    for l in range(L):
        agg = _sc_gine(hflat, ee, gidx_src, dsta, zeros_np)
        h2 = hflat[:2 * N].reshape(2, N, 128)
        eps1 = (1.0 + gin_eps[l]).reshape(1, 1)
        hnew = _tc_gin_mlp(h2, agg, gW1[l], gB1[l].reshape(1, H),
                           gW2[l], gB2[l].reshape(1, H), eps1)
        hflat = hnew.reshape(2 * N, 128)

    hc = _sc_hcomb(hflat, gidx_src128, gidx_dst)

    sigma1 = jnp.maximum(sigma, 1e-30).reshape(1, 1).astype(f32)
    w1a = oW1[:H]
    w1b = oW1[H:2 * H]
    w1c = oW1[2 * H:]
    b1eff = _tc_sigma_prep(sigma.reshape(1, 1), sW1, sB1.reshape(1, H),
                           sW2, sB2.reshape(1, H), w1c, oB1.reshape(1, H))
    oW3p = jnp.zeros((H // 2, 128), f32).at[:, 0:1].set(oW3)
    oB3p = jnp.zeros((1, 128), f32).at[0, 0].set(oB3[0])
    emf = edge_mask.astype(f32).reshape(E, 1)

    out = _tc_final_mlp(hc, de, emf, w1a, w1b, b1eff,
                        oW2, oB2.reshape(1, H // 2), oW3p, oB3p,
                        sigma.reshape(1, 1))
    return out[:, 0]


# speedup vs baseline: 1.3733x; 1.3733x over previous
import functools

import jax
import jax.numpy as jnp
from jax import lax
from jax.experimental import pallas as pl
from jax.experimental.pallas import tpu as pltpu
from jax.experimental.pallas import tpu_sc as plsc

N = 10000
E = 160000
H = 256
L = 10
NE = 119
NET = 7

NTILES = 32
EP = 163840
EC = EP // NTILES
B = 128
NCHUNK = EC // B
ECS = EP // 16
NCHUNKS = ECS // B
BG = 64
NCB = 40
NPHASE = ECS // (BG * NCB)
NCG = NPHASE * NCB
NP = 10112
NPT = NP // 16

_MESH = plsc.VectorSubcoreMesh(core_axis_name="c", subcore_axis_name="s")


def _sc_prep(ntab, nidx, posx, posy, posz, srcr, dstr):
    @functools.partial(
        pl.kernel,
        mesh=_MESH,
        out_type=[
            jax.ShapeDtypeStruct((20480, 128), jnp.float32),
            jax.ShapeDtypeStruct((NTILES, NCHUNK, B), jnp.float32),
        ],
        scratch_types=[
            pltpu.VMEM((5, 128), jnp.int32),
            pltpu.VMEM((128, 128), jnp.float32),
            pltpu.VMEM((NCHUNK, B), jnp.int32),
            pltpu.VMEM((NCHUNK, B), jnp.int32),
            pltpu.VMEM((6, B), jnp.float32),
            pltpu.VMEM((B,), jnp.float32),
            pltpu.SemaphoreType.DMA,
        ],
    )
    def k(ntab_hbm, nidx_hbm, px_hbm, py_hbm, pz_hbm, srcr_hbm, dstr_hbm,
          h0_out, d2_out, nidx_v, nrows_v, sidx_v, didx_v, xyz_v, d2_v, sem):
        c = lax.axis_index("c")
        s = lax.axis_index("s")
        w = s * 2 + c

        pltpu.sync_copy(nidx_hbm.at[w], nidx_v)
        for j in range(5):
            pltpu.async_copy(ntab_hbm.at[nidx_v.at[j]], nrows_v, sem).wait()
            pltpu.sync_copy(nrows_v, h0_out.at[pl.ds(w * 640 + j * 128, 128)])

        pltpu.sync_copy(srcr_hbm.at[w], sidx_v)
        pltpu.sync_copy(dstr_hbm.at[w], didx_v)

        def chunk(kk, _):
            for comp, p_hbm in enumerate((px_hbm, py_hbm, pz_hbm)):
                pltpu.async_copy(p_hbm.at[sidx_v.at[kk]],
                                 xyz_v.at[comp], sem).wait()
                pltpu.async_copy(p_hbm.at[didx_v.at[kk]],
                                 xyz_v.at[3 + comp], sem).wait()

            def vec(v, _):
                sl = pl.ds(v * 16, 16)
                dx = xyz_v[0, sl] - xyz_v[3, sl]
                dy = xyz_v[1, sl] - xyz_v[4, sl]
                dz = xyz_v[2, sl] - xyz_v[5, sl]
                d2_v[sl] = dx * dx + dy * dy + dz * dz
                return 0

            lax.fori_loop(0, B // 16, vec, 0, unroll=True)
            pltpu.sync_copy(d2_v, d2_out.at[w, kk])
            return 0

        lax.fori_loop(0, NCHUNK, chunk, 0)

    return k(ntab, nidx, posx, posy, posz, srcr, dstr)


def _sc_gine(hcat, ee, gidx, dsta, zeros_np):
    @functools.partial(
        pl.kernel,
        mesh=_MESH,
        out_type=jax.ShapeDtypeStruct((2, NP, 128), jnp.float32),
        scratch_types=[
            pltpu.VMEM((NCB, BG), jnp.int32),
            pltpu.VMEM((NCB, BG), jnp.int32),
            pltpu.VMEM((2, BG, 128), jnp.float32),
            pltpu.VMEM((2, BG, 128), jnp.float32),
            pltpu.VMEM_SHARED((NP, 128), jnp.float32),
            pltpu.SemaphoreType.DMA,
            pltpu.SemaphoreType.DMA,
            pltpu.SemaphoreType.DMA,
            pltpu.SemaphoreType.DMA,
            pltpu.SemaphoreType.DMA,
            pltpu.SemaphoreType.DMA,
        ],
    )
    def k(hcat_hbm, ee_hbm, gidx_hbm, dsta_hbm, zeros_hbm, agg_out,
          gidx_v, dst_v, hbuf, eebuf, aggsh, g0, g1, e0, e1, s0, s1):
        c = lax.axis_index("c")
        s = lax.axis_index("s")

        pltpu.sync_copy(zeros_hbm.at[pl.ds(s * NPT, NPT)],
                        aggsh.at[pl.ds(s * NPT, NPT)])
        plsc.subcore_barrier()

        def issue_g(p, kk, slot, gsem):
            pltpu.async_copy(hcat_hbm.at[gidx_v.at[kk]], hbuf.at[slot], gsem)

        def issue_e(p, kk, slot, esem):
            pltpu.async_copy(
                ee_hbm.at[c, pl.ds(s * ECS + (p * NCB + kk) * BG, BG)],
                eebuf.at[slot], esem)

        def wait_loads(slot, gsem, esem):
            pltpu.make_async_copy(hcat_hbm.at[gidx_v.at[0]],
                                  hbuf.at[slot], gsem).wait()
            pltpu.make_async_copy(ee_hbm.at[c, pl.ds(0, BG)],
                                  eebuf.at[slot], esem).wait()

        def wait_scatter(slot, ssem):
            pltpu.make_async_copy(ee_hbm.at[c, pl.ds(0, BG)],
                                  eebuf.at[slot], ssem).wait()

        def compute_scatter(kk, slot, ssem):
            def edge(e, _):
                for v in range(8):
                    sl = pl.ds(v * 16, 16)
                    eebuf[slot, e, sl] = jnp.maximum(
                        hbuf[slot, e, sl] + eebuf[slot, e, sl], 0.0)
                return 0

            lax.fori_loop(0, BG, edge, 0)
            pltpu.async_copy(eebuf.at[slot], aggsh.at[dst_v.at[kk]],
                             ssem, add=True)

        def phase(p, _):
            @pl.when(p > 0)
            def _():
                wait_scatter(0, s0)
                wait_scatter(1, s1)

            pltpu.sync_copy(gidx_hbm.at[c, s, pl.ds(p * NCB, NCB)], gidx_v)
            pltpu.sync_copy(dsta_hbm.at[s, pl.ds(p * NCB, NCB)], dst_v)
            issue_g(p, 0, 0, g0)
            issue_e(p, 0, 0, e0)
            issue_g(p, 1, 1, g1)

            def pair(j, _):
                @pl.when(j > 0)
                def _():
                    wait_scatter(1, s1)
                issue_e(p, 2 * j + 1, 1, e1)
                wait_loads(0, g0, e0)
                compute_scatter(2 * j, 0, s0)

                @pl.when(j < NCB // 2 - 1)
                def _():
                    issue_g(p, 2 * j + 2, 0, g0)

                @pl.when(j < NCB // 2 - 1)
                def _():
                    wait_scatter(0, s0)
                    issue_e(p, 2 * j + 2, 0, e0)
                wait_loads(1, g1, e1)
                compute_scatter(2 * j + 1, 1, s1)

                @pl.when(j < NCB // 2 - 1)
                def _():
                    issue_g(p, 2 * j + 3, 1, g1)
                return 0

            lax.fori_loop(0, NCB // 2, pair, 0)
            return 0

        lax.fori_loop(0, NPHASE, phase, 0)
        wait_scatter(0, s0)
        wait_scatter(1, s1)
        plsc.subcore_barrier()
        pltpu.sync_copy(aggsh.at[pl.ds(s * NPT, NPT)],
                        agg_out.at[c, pl.ds(s * NPT, NPT)])

    return k(hcat, ee, gidx, dsta, zeros_np)


def _sc_hcomb(hcat, gsrc, gdst):
    @functools.partial(
        pl.kernel,
        mesh=_MESH,
        out_type=jax.ShapeDtypeStruct((2, EP, 128), jnp.float32),
        scratch_types=[
            pltpu.VMEM((NCHUNKS, B), jnp.int32),
            pltpu.VMEM((NCHUNKS, B), jnp.int32),
            pltpu.VMEM((2, B, 128), jnp.float32),
            pltpu.VMEM((2, B, 128), jnp.float32),
            pltpu.VMEM((2, B, 128), jnp.float32),
            pltpu.SemaphoreType.DMA,
            pltpu.SemaphoreType.DMA,
            pltpu.SemaphoreType.DMA,
            pltpu.SemaphoreType.DMA,
            pltpu.SemaphoreType.DMA,
            pltpu.SemaphoreType.DMA,
        ],
    )
    def k(hcat_hbm, gsrc_hbm, gdst_hbm, hc_out,
          gs_v, gd_v, abuf, bbuf, obuf, a0, a1, b0, b1, w0, w1):
        c = lax.axis_index("c")
        s = lax.axis_index("s")
        asems = (a0, a1)
        bsems = (b0, b1)
        wsems = (w0, w1)

        pltpu.sync_copy(gsrc_hbm.at[c, s], gs_v)
        pltpu.sync_copy(gdst_hbm.at[c, s], gd_v)

        def issue(kk, slot):
            pltpu.async_copy(hcat_hbm.at[gs_v.at[kk]], abuf.at[slot],
                             asems[slot])
            pltpu.async_copy(hcat_hbm.at[gd_v.at[kk]], bbuf.at[slot],
                             bsems[slot])

        def wait_loads(slot):
            pltpu.make_async_copy(hcat_hbm.at[gs_v.at[0]], abuf.at[slot],
                                  asems[slot]).wait()
            pltpu.make_async_copy(hcat_hbm.at[gd_v.at[0]], bbuf.at[slot],
                                  bsems[slot]).wait()

        def wait_write(slot):
            pltpu.make_async_copy(hc_out.at[c, pl.ds(0, B)], obuf.at[slot],
                                  wsems[slot]).wait()

        def compute_write(kk, slot):
            def edge(e, _):
                for v in range(8):
                    sl = pl.ds(v * 16, 16)
                    obuf[slot, e, sl] = abuf[slot, e, sl] * bbuf[slot, e, sl]
                return 0

            lax.fori_loop(0, B, edge, 0)
            pltpu.async_copy(obuf.at[slot],
                             hc_out.at[c, pl.ds(s * ECS + kk * B, B)],
                             wsems[slot])

        issue(0, 0)
        issue(1, 1)

        def pair(j, _):
            @pl.when(j > 0)
            def _():
                issue(2 * j + 1, 1)

            wait_loads(0)

            @pl.when(j > 0)
            def _():
                wait_write(0)

            compute_write(2 * j, 0)

            @pl.when(j < NCHUNKS // 2 - 1)
            def _():
                issue(2 * j + 2, 0)

            wait_loads(1)

            @pl.when(j > 0)
            def _():
                wait_write(1)

            compute_write(2 * j + 1, 1)
            return 0

        lax.fori_loop(0, NCHUNKS // 2, pair, 0)
        wait_write(0)
        wait_write(1)

    return k(hcat, gsrc, gdst)


BE2 = 2048


def _tc_edge_prep(dist2, et, etab, dW1, dB1, dW2, dB2):
    def body(d2_ref, et_ref, etab_ref, w1_ref, b1_ref, w2_ref, b2_ref,
             de_ref, ee_ref):
        d = jnp.sqrt(d2_ref[...])
        h1 = jnp.maximum(d * w1_ref[...] + b1_ref[...], 0.0)
        de = jax.lax.dot(h1, w2_ref[...],
                         preferred_element_type=jnp.float32) + b2_ref[...]
        et = et_ref[...]
        sel = jnp.zeros_like(de)
        for t in range(NET):
            sel = sel + jnp.where(et == t, 1.0, 0.0) * etab_ref[t:t + 1, :]
        ee = sel * de
        de_ref[...] = de
        ee_ref[0] = ee[:, :128]
        ee_ref[1] = ee[:, 128:]

    return pl.pallas_call(
        body,
        grid=(EP // BE2,),
        in_specs=[
            pl.BlockSpec((BE2, 1), lambda i: (i, 0)),
            pl.BlockSpec((BE2, 1), lambda i: (i, 0)),
            pl.BlockSpec((8, H), lambda i: (0, 0)),
            pl.BlockSpec((1, H), lambda i: (0, 0)),
            pl.BlockSpec((1, H), lambda i: (0, 0)),
            pl.BlockSpec((H, H), lambda i: (0, 0)),
            pl.BlockSpec((1, H), lambda i: (0, 0)),
        ],
        out_specs=[
            pl.BlockSpec((BE2, H), lambda i: (i, 0)),
            pl.BlockSpec((2, BE2, 128), lambda i: (0, i, 0)),
        ],
        out_shape=[
            jax.ShapeDtypeStruct((EP, H), jnp.float32),
            jax.ShapeDtypeStruct((2, EP, 128), jnp.float32),
        ],
    )(dist2, et, etab, dW1, dB1, dW2, dB2)


BN = 2000


def _tc_gin_mlp(h, agg, w1, b1, w2, b2, eps1):
    def body(h_ref, agg_ref, w1_ref, b1_ref, w2_ref, b2_ref, eps_ref, out_ref):
        e = eps_ref[0, 0]
        x0 = e * h_ref[0] + agg_ref[0]
        x1 = e * h_ref[1] + agg_ref[1]
        hm = jnp.maximum(
            jax.lax.dot(x0, w1_ref[...][:128],
                        preferred_element_type=jnp.float32)
            + jax.lax.dot(x1, w1_ref[...][128:],
                          preferred_element_type=jnp.float32)
            + b1_ref[...], 0.0)
        y = jax.lax.dot(hm, w2_ref[...],
                        preferred_element_type=jnp.float32) + b2_ref[...]
        out_ref[0] = y[:, :128]
        out_ref[1] = y[:, 128:]

    return pl.pallas_call(
        body,
        grid=(N // BN,),
        in_specs=[
            pl.BlockSpec((2, BN, 128), lambda i: (0, i, 0)),
            pl.BlockSpec((2, BN, 128), lambda i: (0, i, 0)),
            pl.BlockSpec((H, H), lambda i: (0, 0)),
            pl.BlockSpec((1, H), lambda i: (0, 0)),
            pl.BlockSpec((H, H), lambda i: (0, 0)),
            pl.BlockSpec((1, H), lambda i: (0, 0)),
            pl.BlockSpec((1, 1), lambda i: (0, 0), memory_space=pltpu.SMEM),
        ],
        out_specs=pl.BlockSpec((2, BN, 128), lambda i: (0, i, 0)),
        out_shape=jax.ShapeDtypeStruct((2, N, 128), jnp.float32),
    )(h, agg, w1, b1, w2, b2, eps1)


def _tc_sigma_prep(sigma1, sW1, sB1, sW2, sB2, oW1c, oB1):
    def body(sg_ref, w1_ref, b1_ref, w2_ref, b2_ref, wc_ref, ob_ref, out_ref):
        s = jnp.log(jnp.maximum(sg_ref[...], 1e-12))
        h1 = jnp.maximum(s * w1_ref[...] + b1_ref[...], 0.0)
        se = jax.lax.dot(h1, w2_ref[...],
                         preferred_element_type=jnp.float32) + b2_ref[...]
        out_ref[...] = ob_ref[...] + jax.lax.dot(
            se, wc_ref[...], preferred_element_type=jnp.float32)

    return pl.pallas_call(
        body,
        out_shape=jax.ShapeDtypeStruct((1, H), jnp.float32),
    )(sigma1, sW1, sB1, sW2, sB2, oW1c, oB1)


BE = 1600


def _tc_final_mlp(hc, de, emf, w1a, w1b, b1eff, oW2, oB2, oW3p, oB3p, sigma1):
    def body(hc_ref, de_ref, emf_ref, w1a_ref, w1b_ref, b1_ref,
             w2_ref, b2_ref, w3_ref, b3_ref, sg_ref, out_ref):
        w1a = w1a_ref[...]
        h1 = jnp.maximum(
            jax.lax.dot(hc_ref[0], w1a[:128], preferred_element_type=jnp.float32)
            + jax.lax.dot(hc_ref[1], w1a[128:], preferred_element_type=jnp.float32)
            + jax.lax.dot(de_ref[...], w1b_ref[...],
                          preferred_element_type=jnp.float32)
            + b1_ref[...], 0.0)
        h2 = jnp.maximum(
            jax.lax.dot(h1, w2_ref[...], preferred_element_type=jnp.float32)
            + b2_ref[...], 0.0)
        raw = jax.lax.dot(h2, w3_ref[...],
                          preferred_element_type=jnp.float32) + b3_ref[...]
        out_ref[...] = raw * emf_ref[...] / sg_ref[0, 0]

    return pl.pallas_call(
        body,
        grid=(E // BE,),
        in_specs=[
            pl.BlockSpec((2, BE, 128), lambda i: (0, i, 0)),
            pl.BlockSpec((BE, H), lambda i: (i, 0)),
            pl.BlockSpec((BE, 1), lambda i: (i, 0)),
            pl.BlockSpec((H, H), lambda i: (0, 0)),
            pl.BlockSpec((H, H), lambda i: (0, 0)),
            pl.BlockSpec((1, H), lambda i: (0, 0)),
            pl.BlockSpec((H, H // 2), lambda i: (0, 0)),
            pl.BlockSpec((1, H // 2), lambda i: (0, 0)),
            pl.BlockSpec((H // 2, 128), lambda i: (0, 0)),
            pl.BlockSpec((1, 128), lambda i: (0, 0)),
            pl.BlockSpec((1, 1), lambda i: (0, 0), memory_space=pltpu.SMEM),
        ],
        out_specs=pl.BlockSpec((BE, 128), lambda i: (i, 0)),
        out_shape=jax.ShapeDtypeStruct((E, 128), jnp.float32),
    )(hc, de, emf, w1a, w1b, b1eff, oW2, oB2, oW3p, oB3p, sigma1)


def kernel(pos, edge_index, atom_type, edge_type, edge_mask, sigma,
           node_table, edge_table, dW1, dB1, dW2, dB2, sW1, sB1, sW2, sB2,
           gin_eps, gW1, gB1, gW2, gB2, oW1, oB1, oW2, oB2, oW3, oB3):
    f32 = jnp.float32
    src, dst = edge_index[0], edge_index[1]
    pad = EP - E
    srcp = jnp.concatenate([src, jnp.zeros((pad,), jnp.int32)])
    dstp = jnp.concatenate([dst, jnp.zeros((pad,), jnp.int32)])
    etp = jnp.concatenate([edge_type, jnp.zeros((pad,), jnp.int32)])
    dst_adj = jnp.concatenate(
        [jnp.where(edge_mask, dst, N), jnp.full((pad,), N, jnp.int32)])

    gidx_src = jnp.stack([srcp, srcp + N]).reshape(2, 16, NCG, BG)
    gidx_dst = jnp.stack([dstp, dstp + N]).reshape(2, 16, NCHUNKS, B)
    gidx_src128 = gidx_src.reshape(2, 16, NCHUNKS, B)
    dsta = dst_adj.reshape(16, NCG, BG)
    srcr = srcp.reshape(NTILES, NCHUNK, B)
    dstr = dstp.reshape(NTILES, NCHUNK, B)
    nidx = jnp.concatenate(
        [atom_type, atom_type + NE,
         jnp.zeros((20480 - 2 * N,), jnp.int32)]).reshape(NTILES, 5, 128)
    ntab_cat = jnp.concatenate([node_table[:, :128], node_table[:, 128:]], 0)
    zeros_np = jnp.zeros((NP, 128), f32)

    h0p, dist2 = _sc_prep(ntab_cat, nidx, pos[:, 0], pos[:, 1], pos[:, 2],
                          srcr, dstr)

    etab = jnp.concatenate([edge_table, jnp.zeros((1, H), f32)], 0)
    de, ee = _tc_edge_prep(dist2.reshape(EP, 1), etp.reshape(EP, 1), etab,
                           dW1, dB1.reshape(1, H), dW2, dB2.reshape(1, H))

    hflat = h0p
    for l in range(L):
        agg = _sc_gine(hflat, ee, gidx_src, dsta, zeros_np)
        h2 = hflat[:2 * N].reshape(2, N, 128)
        eps1 = (1.0 + gin_eps[l]).reshape(1, 1)
        hnew = _tc_gin_mlp(h2, agg, gW1[l], gB1[l].reshape(1, H),
                           gW2[l], gB2[l].reshape(1, H), eps1)
        hflat = hnew.reshape(2 * N, 128)

    hc = _sc_hcomb(hflat, gidx_src128, gidx_dst)

    sigma1 = jnp.maximum(sigma, 1e-30).reshape(1, 1).astype(f32)
    w1a = oW1[:H]
    w1b = oW1[H:2 * H]
    w1c = oW1[2 * H:]
    b1eff = _tc_sigma_prep(sigma.reshape(1, 1), sW1, sB1.reshape(1, H),
                           sW2, sB2.reshape(1, H), w1c, oB1.reshape(1, H))
    oW3p = jnp.zeros((H // 2, 128), f32).at[:, 0:1].set(oW3)
    oB3p = jnp.zeros((1, 128), f32).at[0, 0].set(oB3[0])
    emf = edge_mask.astype(f32).reshape(E, 1)

    out = _tc_final_mlp(hc, de, emf, w1a, w1b, b1eff,
                        oW2, oB2.reshape(1, H // 2), oW3p, oB3p,
                        sigma.reshape(1, 1))
    return out[:, 0]
